# baseline (speedup 1.0000x reference)
import functools

import jax
import jax.numpy as jnp
from jax import lax
from jax.experimental import pallas as pl
from jax.experimental.pallas import tpu as pltpu
from jax.experimental.pallas import tpu_sc as plsc

N = 10000
NPAD = 10240
E_RAW = 320000
E_TOT = E_RAW + N
NTILES = 32
EPAD = 331776
PAD1D = EPAD + 12800
RPT = NPAD // 16

_MESH = plsc.VectorSubcoreMesh(core_axis_name="c", subcore_axis_name="s")

_SPLIT = {128: (64, 197), 64: (128, 100), 32: (128, 93), 16: (128, 86)}


def _sc_degree(dst_r):
    nmax = 100

    @functools.partial(
        pl.kernel,
        out_type=jax.ShapeDtypeStruct((2, NPAD, 16), jnp.float32),
        mesh=_MESH,
        compiler_params=pltpu.CompilerParams(use_tc_tiling_on_sc=False),
        scratch_types=[
            pltpu.VMEM((nmax, 128), jnp.int32),
            pltpu.VMEM((128, 16), jnp.float32),
            pltpu.VMEM_SHARED((NPAD, 16), jnp.float32),
            pltpu.SemaphoreType.DMA,
        ],
    )
    def k(dst_hbm, out_hbm, idx_v, ones_v, acc, isem):
        c = lax.axis_index("c")
        s = lax.axis_index("s")
        start = s * 162 + c * 81
        pltpu.async_copy(dst_hbm.at[pl.ds(start, nmax)], idx_v, isem)

        def fill(val):
            def body(j, _):
                ones_v[j, :] = jnp.full((16,), val, jnp.float32)
                return 0
            return lax.fori_loop(0, 128, body, 0)

        fill(0.0)
        for t in range(RPT // 128):
            pltpu.sync_copy(ones_v, acc.at[pl.ds(s * RPT + t * 128, 128)])
        fill(1.0)
        plsc.subcore_barrier()
        pltpu.make_async_copy(dst_hbm.at[pl.ds(start, nmax)], idx_v, isem).wait()

        def body(j, _):
            pltpu.async_copy(ones_v, acc.at[idx_v.at[j]], isem, add=True)
            return 0

        lax.fori_loop(0, 81, body, 0)

        def drain(j, _):
            pltpu.make_async_copy(ones_v, acc.at[idx_v.at[0]], isem).wait()
            return 0

        lax.fori_loop(0, 81, drain, 0)
        plsc.subcore_barrier()
        pltpu.sync_copy(acc.at[pl.ds(s * RPT, RPT)],
                        out_hbm.at[c, pl.ds(s * RPT, RPT)])

    return k(dst_r)


def _sc_propagate(g, src_f, dst_f, width):
    chunk, n0 = _SPLIT[width]
    t16 = EPAD // chunk // 16
    n1 = t16 - n0
    nmax = 197 if chunk == 64 else 100

    @functools.partial(
        pl.kernel,
        out_type=jax.ShapeDtypeStruct((2, NPAD, width), jnp.float32),
        mesh=_MESH,
        compiler_params=pltpu.CompilerParams(use_tc_tiling_on_sc=False),
        scratch_types=[
            pltpu.VMEM((nmax, chunk), jnp.int32),
            pltpu.VMEM((nmax, chunk), jnp.int32),
            pltpu.VMEM((2, chunk, width), jnp.float32),
            pltpu.VMEM_SHARED((NPAD, width), jnp.float32),
            pltpu.SemaphoreType.DMA((2,)),
            pltpu.SemaphoreType.DMA((2,)),
            pltpu.SemaphoreType.DMA,
        ],
    )
    def k(g_hbm, src_hbm, dst_hbm, out_hbm, src_v, dst_v, rows_v, acc, gsem,
          ssem, isem):
        c = lax.axis_index("c")
        s = lax.axis_index("s")
        start = s * t16 + c * n0
        cnt = jnp.where(c == 0, n0, n1)
        pltpu.async_copy(src_hbm.at[pl.ds(start, nmax)], src_v, isem)
        pltpu.async_copy(dst_hbm.at[pl.ds(start, nmax)], dst_v, isem)

        zeros16 = jnp.zeros((16,), jnp.float32)

        def zbody(i, _):
            for kk in range(width // 16):
                rows_v[0, i, pl.ds(kk * 16, 16)] = zeros16
            return 0

        lax.fori_loop(0, chunk, zbody, 0)
        for t in range(RPT // chunk):
            pltpu.sync_copy(rows_v.at[0], acc.at[pl.ds(s * RPT + t * chunk, chunk)])
        plsc.subcore_barrier()
        pltpu.make_async_copy(src_hbm.at[pl.ds(start, nmax)], src_v, isem).wait()
        pltpu.make_async_copy(dst_hbm.at[pl.ds(start, nmax)], dst_v, isem).wait()

        pltpu.async_copy(g_hbm.at[src_v.at[0]], rows_v.at[0], gsem.at[0])

        def body(j, _):
            par = lax.rem(j, 2)
            npar = lax.rem(j + 1, 2)

            @pl.when(j >= 1)
            def _():
                pltpu.make_async_copy(rows_v.at[npar], acc.at[dst_v.at[0]],
                                      ssem.at[npar]).wait()

            @pl.when(j < cnt - 1)
            def _():
                pltpu.async_copy(g_hbm.at[src_v.at[j + 1]], rows_v.at[npar],
                                 gsem.at[npar])

            pltpu.make_async_copy(g_hbm.at[src_v.at[j]], rows_v.at[par],
                                  gsem.at[par]).wait()
            pltpu.async_copy(rows_v.at[par], acc.at[dst_v.at[j]],
                             ssem.at[par], add=True)
            return 0

        lax.fori_loop(0, cnt, body, 0)
        last = lax.rem(cnt - 1, 2)
        pltpu.make_async_copy(rows_v.at[last], acc.at[dst_v.at[0]],
                              ssem.at[last]).wait()
        plsc.subcore_barrier()
        pltpu.sync_copy(acc.at[pl.ds(s * RPT, RPT)],
                        out_hbm.at[c, pl.ds(s * RPT, RPT)])

    return k(g, src_f, dst_f)


def _k0_body(dp_ref, x_ref, w_ref, g_ref, dis_ref):
    deg = (dp_ref[0] + dp_ref[1])[:, 0:1]
    row = lax.broadcasted_iota(jnp.int32, (NPAD, 1), 0)
    dis = jnp.where((deg > 0) & (row < N), lax.rsqrt(deg), 0.0)
    g_ref[...] = jnp.dot(x_ref[...], w_ref[...],
                         preferred_element_type=jnp.float32) * dis
    dis_ref[...] = dis


def _kmid_body(p_ref, dis_ref, b_ref, w_ref, g_ref):
    dis = dis_ref[...]
    z = jnp.maximum((p_ref[0] + p_ref[1]) * dis + b_ref[...], 0.0)
    g_ref[...] = jnp.dot(z, w_ref[...], preferred_element_type=jnp.float32) * dis


def _k4_body(p_ref, dis_ref, b_ref, g_ref):
    dis = dis_ref[...]
    g_ref[...] = jnp.maximum((p_ref[0] + p_ref[1]) * dis + b_ref[...], 0.0) * dis


def _k5_body(p_ref, dis_ref, b_ref, w_ref, o_ref):
    o_ref[...] = jnp.dot((p_ref[0] + p_ref[1]) * dis_ref[...], w_ref[...],
                         preferred_element_type=jnp.float32) + b_ref[...]


def _views(flat1d, chunk, nmax):
    rows = EPAD // chunk + nmax
    return lax.slice(flat1d, (0,), (rows * chunk,)).reshape(rows, chunk)


def kernel(x, edge_index, W1, b1, W2, b2, W3, b3, W4, b4, W5, b5):
    f32 = jnp.float32
    xp = jnp.concatenate([x, jnp.zeros((NPAD - N, x.shape[1]), f32)], axis=0)
    loop = jnp.arange(N, dtype=jnp.int32)
    padv = jnp.full((PAD1D - E_TOT,), N, dtype=jnp.int32)
    src1d = jnp.concatenate([edge_index[0].astype(jnp.int32), loop, padv])
    dst1d = jnp.concatenate([edge_index[1].astype(jnp.int32), loop, padv])
    src128, dst128 = _views(src1d, 128, 100), _views(dst1d, 128, 100)
    src_w = {128: _views(src1d, 64, 197), 64: src128, 32: src128, 16: src128}
    dst_w = {128: _views(dst1d, 64, 197), 64: dst128, 32: dst128, 16: dst128}

    deg_parts = _sc_degree(dst128)
    g, dis = pl.pallas_call(_k0_body, out_shape=(
        jax.ShapeDtypeStruct((NPAD, 128), f32),
        jax.ShapeDtypeStruct((NPAD, 1), f32)))(deg_parts, xp, W1)

    p = _sc_propagate(g, src_w[128], dst_w[128], 128)
    g = pl.pallas_call(_kmid_body, out_shape=jax.ShapeDtypeStruct(
        (NPAD, 64), f32))(p, dis, b1.reshape(1, -1), W2)
    p = _sc_propagate(g, src_w[64], dst_w[64], 64)
    g = pl.pallas_call(_kmid_body, out_shape=jax.ShapeDtypeStruct(
        (NPAD, 32), f32))(p, dis, b2.reshape(1, -1), W3)
    p = _sc_propagate(g, src_w[32], dst_w[32], 32)
    g = pl.pallas_call(_kmid_body, out_shape=jax.ShapeDtypeStruct(
        (NPAD, 16), f32))(p, dis, b3.reshape(1, -1), W4)
    p = _sc_propagate(g, src_w[16], dst_w[16], 16)
    g = pl.pallas_call(_k4_body, out_shape=jax.ShapeDtypeStruct(
        (NPAD, 16), f32))(p, dis, b4.reshape(1, -1))
    p = _sc_propagate(g, src_w[16], dst_w[16], 16)
    out = pl.pallas_call(_k5_body, out_shape=jax.ShapeDtypeStruct(
        (NPAD, 40), f32))(p, dis, b5.reshape(1, -1), W5)
    return out[:N]


# speedup vs baseline: 1.0078x; 1.0078x over previous
import functools

import jax
import jax.numpy as jnp
from jax import lax
from jax.experimental import pallas as pl
from jax.experimental.pallas import tpu as pltpu
from jax.experimental.pallas import tpu_sc as plsc

N = 10000
NPAD = 10240
E_RAW = 320000
E_TOT = E_RAW + N
NTILES = 32
EPAD = 331776
PAD1D = EPAD + 13120
RPT = NPAD // 16

_MESH = plsc.VectorSubcoreMesh(core_axis_name="c", subcore_axis_name="s")

_SPLIT = {128: (64, 205), 64: (128, 98), 32: (128, 95), 16: (128, 86)}


def _sc_degree(dst_r):
    nmax = 100

    @functools.partial(
        pl.kernel,
        out_type=jax.ShapeDtypeStruct((2, NPAD, 16), jnp.float32),
        mesh=_MESH,
        compiler_params=pltpu.CompilerParams(use_tc_tiling_on_sc=False),
        scratch_types=[
            pltpu.VMEM((nmax, 128), jnp.int32),
            pltpu.VMEM((128, 16), jnp.float32),
            pltpu.VMEM_SHARED((NPAD, 16), jnp.float32),
            pltpu.SemaphoreType.DMA,
        ],
    )
    def k(dst_hbm, out_hbm, idx_v, ones_v, acc, isem):
        c = lax.axis_index("c")
        s = lax.axis_index("s")
        start = s * 162 + c * 81
        pltpu.async_copy(dst_hbm.at[pl.ds(start, nmax)], idx_v, isem)

        def fill(val):
            def body(j, _):
                ones_v[j, :] = jnp.full((16,), val, jnp.float32)
                return 0
            return lax.fori_loop(0, 128, body, 0)

        fill(0.0)
        for t in range(RPT // 128):
            pltpu.sync_copy(ones_v, acc.at[pl.ds(s * RPT + t * 128, 128)])
        fill(1.0)
        plsc.subcore_barrier()
        pltpu.make_async_copy(dst_hbm.at[pl.ds(start, nmax)], idx_v, isem).wait()

        def body(j, _):
            pltpu.async_copy(ones_v, acc.at[idx_v.at[j]], isem, add=True)
            return 0

        lax.fori_loop(0, 81, body, 0)

        def drain(j, _):
            pltpu.make_async_copy(ones_v, acc.at[idx_v.at[0]], isem).wait()
            return 0

        lax.fori_loop(0, 81, drain, 0)
        plsc.subcore_barrier()
        pltpu.sync_copy(acc.at[pl.ds(s * RPT, RPT)],
                        out_hbm.at[c, pl.ds(s * RPT, RPT)])

    return k(dst_r)


def _sc_propagate(g, src_f, dst_f, width):
    chunk, n0 = _SPLIT[width]
    t16 = EPAD // chunk // 16
    n1 = t16 - n0
    nmax = 205 if chunk == 64 else 100

    @functools.partial(
        pl.kernel,
        out_type=jax.ShapeDtypeStruct((2, NPAD, width), jnp.float32),
        mesh=_MESH,
        compiler_params=pltpu.CompilerParams(use_tc_tiling_on_sc=False),
        scratch_types=[
            pltpu.VMEM((nmax, chunk), jnp.int32),
            pltpu.VMEM((nmax, chunk), jnp.int32),
            pltpu.VMEM((2, chunk, width), jnp.float32),
            pltpu.VMEM_SHARED((NPAD, width), jnp.float32),
            pltpu.SemaphoreType.DMA((2,)),
            pltpu.SemaphoreType.DMA((2,)),
            pltpu.SemaphoreType.DMA,
        ],
    )
    def k(g_hbm, src_hbm, dst_hbm, out_hbm, src_v, dst_v, rows_v, acc, gsem,
          ssem, isem):
        c = lax.axis_index("c")
        s = lax.axis_index("s")
        start = s * t16 + c * n0
        cnt = jnp.where(c == 0, n0, n1)
        pltpu.async_copy(src_hbm.at[pl.ds(start, nmax)], src_v, isem)
        pltpu.async_copy(dst_hbm.at[pl.ds(start, nmax)], dst_v, isem)

        zeros16 = jnp.zeros((16,), jnp.float32)

        def zbody(i, _):
            for kk in range(width // 16):
                rows_v[0, i, pl.ds(kk * 16, 16)] = zeros16
            return 0

        lax.fori_loop(0, chunk, zbody, 0)
        for t in range(RPT // chunk):
            pltpu.sync_copy(rows_v.at[0], acc.at[pl.ds(s * RPT + t * chunk, chunk)])
        plsc.subcore_barrier()
        pltpu.make_async_copy(src_hbm.at[pl.ds(start, nmax)], src_v, isem).wait()
        pltpu.make_async_copy(dst_hbm.at[pl.ds(start, nmax)], dst_v, isem).wait()

        pltpu.async_copy(g_hbm.at[src_v.at[0]], rows_v.at[0], gsem.at[0])

        def body(j, _):
            par = lax.rem(j, 2)
            npar = lax.rem(j + 1, 2)

            if width <= 32:
                @pl.when(j >= 1)
                def _():
                    pltpu.make_async_copy(rows_v.at[npar], acc.at[dst_v.at[0]],
                                          ssem.at[npar]).wait()

            @pl.when(j < cnt - 1)
            def _():
                pltpu.async_copy(g_hbm.at[src_v.at[j + 1]], rows_v.at[npar],
                                 gsem.at[npar])

            pltpu.make_async_copy(g_hbm.at[src_v.at[j]], rows_v.at[par],
                                  gsem.at[par]).wait()
            if width <= 32:
                pltpu.async_copy(rows_v.at[par], acc.at[dst_v.at[j]],
                                 ssem.at[par], add=True)
            else:
                pltpu.sync_copy(rows_v.at[par], acc.at[dst_v.at[j]], add=True)
            return 0

        lax.fori_loop(0, cnt, body, 0)
        if width <= 32:
            last = lax.rem(cnt - 1, 2)
            pltpu.make_async_copy(rows_v.at[last], acc.at[dst_v.at[0]],
                                  ssem.at[last]).wait()
        plsc.subcore_barrier()
        pltpu.sync_copy(acc.at[pl.ds(s * RPT, RPT)],
                        out_hbm.at[c, pl.ds(s * RPT, RPT)])

    return k(g, src_f, dst_f)


def _k0_body(dp_ref, x_ref, w_ref, g_ref, dis_ref):
    deg = (dp_ref[0] + dp_ref[1])[:, 0:1]
    row = lax.broadcasted_iota(jnp.int32, (NPAD, 1), 0)
    dis = jnp.where((deg > 0) & (row < N), lax.rsqrt(deg), 0.0)
    g_ref[...] = jnp.dot(x_ref[...], w_ref[...],
                         preferred_element_type=jnp.float32) * dis
    dis_ref[...] = dis


def _kmid_body(p_ref, dis_ref, b_ref, w_ref, g_ref):
    dis = dis_ref[...]
    z = jnp.maximum((p_ref[0] + p_ref[1]) * dis + b_ref[...], 0.0)
    g_ref[...] = jnp.dot(z, w_ref[...], preferred_element_type=jnp.float32) * dis


def _k4_body(p_ref, dis_ref, b_ref, g_ref):
    dis = dis_ref[...]
    g_ref[...] = jnp.maximum((p_ref[0] + p_ref[1]) * dis + b_ref[...], 0.0) * dis


def _k5_body(p_ref, dis_ref, b_ref, w_ref, o_ref):
    o_ref[...] = jnp.dot((p_ref[0] + p_ref[1]) * dis_ref[...], w_ref[...],
                         preferred_element_type=jnp.float32) + b_ref[...]


def _views(flat1d, chunk, nmax):
    rows = EPAD // chunk + nmax
    return lax.slice(flat1d, (0,), (rows * chunk,)).reshape(rows, chunk)


def kernel(x, edge_index, W1, b1, W2, b2, W3, b3, W4, b4, W5, b5):
    f32 = jnp.float32
    xp = jnp.concatenate([x, jnp.zeros((NPAD - N, x.shape[1]), f32)], axis=0)
    loop = jnp.arange(N, dtype=jnp.int32)
    padv = jnp.full((PAD1D - E_TOT,), N, dtype=jnp.int32)
    src1d = jnp.concatenate([edge_index[0].astype(jnp.int32), loop, padv])
    dst1d = jnp.concatenate([edge_index[1].astype(jnp.int32), loop, padv])
    src128, dst128 = _views(src1d, 128, 100), _views(dst1d, 128, 100)
    src_w = {128: _views(src1d, 64, 205), 64: src128, 32: src128, 16: src128}
    dst_w = {128: _views(dst1d, 64, 205), 64: dst128, 32: dst128, 16: dst128}

    deg_parts = _sc_degree(dst128)
    g, dis = pl.pallas_call(_k0_body, out_shape=(
        jax.ShapeDtypeStruct((NPAD, 128), f32),
        jax.ShapeDtypeStruct((NPAD, 1), f32)))(deg_parts, xp, W1)

    p = _sc_propagate(g, src_w[128], dst_w[128], 128)
    g = pl.pallas_call(_kmid_body, out_shape=jax.ShapeDtypeStruct(
        (NPAD, 64), f32))(p, dis, b1.reshape(1, -1), W2)
    p = _sc_propagate(g, src_w[64], dst_w[64], 64)
    g = pl.pallas_call(_kmid_body, out_shape=jax.ShapeDtypeStruct(
        (NPAD, 32), f32))(p, dis, b2.reshape(1, -1), W3)
    p = _sc_propagate(g, src_w[32], dst_w[32], 32)
    g = pl.pallas_call(_kmid_body, out_shape=jax.ShapeDtypeStruct(
        (NPAD, 16), f32))(p, dis, b3.reshape(1, -1), W4)
    p = _sc_propagate(g, src_w[16], dst_w[16], 16)
    g = pl.pallas_call(_k4_body, out_shape=jax.ShapeDtypeStruct(
        (NPAD, 16), f32))(p, dis, b4.reshape(1, -1))
    p = _sc_propagate(g, src_w[16], dst_w[16], 16)
    out = pl.pallas_call(_k5_body, out_shape=jax.ShapeDtypeStruct(
        (NPAD, 40), f32))(p, dis, b5.reshape(1, -1), W5)
    return out[:N]


# speedup vs baseline: 1.1314x; 1.1226x over previous
import functools

import jax
import jax.numpy as jnp
from jax import lax
from jax.experimental import pallas as pl
from jax.experimental.pallas import tpu as pltpu
from jax.experimental.pallas import tpu_sc as plsc

N = 10000
NPAD = 10240
E_RAW = 320000
E_TOT = E_RAW + N
NTILES = 32
EPAD = 331776
PAD1D = EPAD + 13120
RPT = NPAD // 16

_MESH = plsc.VectorSubcoreMesh(core_axis_name="c", subcore_axis_name="s")

_SPLIT = {128: (64, 205), 64: (128, 98), 32: (128, 95), 16: (128, 86)}


def _sc_degree(dst_r):
    nmax = 100

    @functools.partial(
        pl.kernel,
        out_type=jax.ShapeDtypeStruct((2, NPAD, 16), jnp.float32),
        mesh=_MESH,
        compiler_params=pltpu.CompilerParams(use_tc_tiling_on_sc=False),
        scratch_types=[
            pltpu.VMEM((nmax, 128), jnp.int32),
            pltpu.VMEM((128, 16), jnp.float32),
            pltpu.VMEM_SHARED((NPAD, 16), jnp.float32),
            pltpu.SemaphoreType.DMA,
        ],
    )
    def k(dst_hbm, out_hbm, idx_v, ones_v, acc, isem):
        c = lax.axis_index("c")
        s = lax.axis_index("s")
        start = s * 162 + c * 81
        pltpu.async_copy(dst_hbm.at[pl.ds(start, nmax)], idx_v, isem)

        def fill(val):
            def body(j, _):
                ones_v[j, :] = jnp.full((16,), val, jnp.float32)
                return 0
            return lax.fori_loop(0, 128, body, 0)

        fill(0.0)
        for t in range(RPT // 128):
            pltpu.sync_copy(ones_v, acc.at[pl.ds(s * RPT + t * 128, 128)])
        fill(1.0)
        plsc.subcore_barrier()
        pltpu.make_async_copy(dst_hbm.at[pl.ds(start, nmax)], idx_v, isem).wait()

        def body(j, _):
            pltpu.async_copy(ones_v, acc.at[idx_v.at[j]], isem, add=True)
            return 0

        lax.fori_loop(0, 81, body, 0)

        def drain(j, _):
            pltpu.make_async_copy(ones_v, acc.at[idx_v.at[0]], isem).wait()
            return 0

        lax.fori_loop(0, 81, drain, 0)
        plsc.subcore_barrier()
        pltpu.sync_copy(acc.at[pl.ds(s * RPT, RPT)],
                        out_hbm.at[c, pl.ds(s * RPT, RPT)])

    return k(dst_r)


def _sc_propagate(g, src_f, dst_f, width):
    chunk, n0 = _SPLIT[width]
    t16 = EPAD // chunk // 16
    n1 = t16 - n0
    nmax = 205 if chunk == 64 else 100
    from_spmem = width <= 64

    @functools.partial(
        pl.kernel,
        out_type=jax.ShapeDtypeStruct((2, NPAD, width), jnp.float32),
        mesh=_MESH,
        compiler_params=pltpu.CompilerParams(use_tc_tiling_on_sc=False),
        scratch_types=[
            pltpu.VMEM((nmax, chunk), jnp.int32),
            pltpu.VMEM((nmax, chunk), jnp.int32),
            pltpu.VMEM((2, chunk, width), jnp.float32),
            pltpu.VMEM_SHARED((NPAD, width), jnp.float32),
            pltpu.VMEM_SHARED((NPAD if from_spmem else 16, width), jnp.float32),
            pltpu.SemaphoreType.DMA((2,)),
            pltpu.SemaphoreType.DMA((2,)),
            pltpu.SemaphoreType.DMA,
        ],
    )
    def k(g_hbm, src_hbm, dst_hbm, out_hbm, src_v, dst_v, rows_v, acc, table,
          gsem, ssem, isem):
        c = lax.axis_index("c")
        s = lax.axis_index("s")
        start = s * t16 + c * n0
        cnt = jnp.where(c == 0, n0, n1)
        pltpu.async_copy(src_hbm.at[pl.ds(start, nmax)], src_v, isem)
        pltpu.async_copy(dst_hbm.at[pl.ds(start, nmax)], dst_v, isem)
        if from_spmem:
            pltpu.async_copy(g_hbm.at[pl.ds(s * RPT, RPT)],
                             table.at[pl.ds(s * RPT, RPT)], isem)
        g_src = table if from_spmem else g_hbm

        zeros16 = jnp.zeros((16,), jnp.float32)

        def zbody(i, _):
            for kk in range(width // 16):
                rows_v[0, i, pl.ds(kk * 16, 16)] = zeros16
            return 0

        lax.fori_loop(0, chunk, zbody, 0)
        for t in range(RPT // chunk):
            pltpu.sync_copy(rows_v.at[0], acc.at[pl.ds(s * RPT + t * chunk, chunk)])
        pltpu.make_async_copy(src_hbm.at[pl.ds(start, nmax)], src_v, isem).wait()
        pltpu.make_async_copy(dst_hbm.at[pl.ds(start, nmax)], dst_v, isem).wait()
        if from_spmem:
            pltpu.make_async_copy(g_hbm.at[pl.ds(s * RPT, RPT)],
                                  table.at[pl.ds(s * RPT, RPT)], isem).wait()
        plsc.subcore_barrier()

        pltpu.async_copy(g_src.at[src_v.at[0]], rows_v.at[0], gsem.at[0])

        def body(j, _):
            par = lax.rem(j, 2)
            npar = lax.rem(j + 1, 2)

            if width <= 32:
                @pl.when(j >= 1)
                def _():
                    pltpu.make_async_copy(rows_v.at[npar], acc.at[dst_v.at[0]],
                                          ssem.at[npar]).wait()

            @pl.when(j < cnt - 1)
            def _():
                pltpu.async_copy(g_src.at[src_v.at[j + 1]], rows_v.at[npar],
                                 gsem.at[npar])

            pltpu.make_async_copy(g_src.at[src_v.at[j]], rows_v.at[par],
                                  gsem.at[par]).wait()
            if width <= 32:
                pltpu.async_copy(rows_v.at[par], acc.at[dst_v.at[j]],
                                 ssem.at[par], add=True)
            else:
                pltpu.sync_copy(rows_v.at[par], acc.at[dst_v.at[j]], add=True)
            return 0

        lax.fori_loop(0, cnt, body, 0)
        if width <= 32:
            last = lax.rem(cnt - 1, 2)
            pltpu.make_async_copy(rows_v.at[last], acc.at[dst_v.at[0]],
                                  ssem.at[last]).wait()
        plsc.subcore_barrier()
        pltpu.sync_copy(acc.at[pl.ds(s * RPT, RPT)],
                        out_hbm.at[c, pl.ds(s * RPT, RPT)])

    return k(g, src_f, dst_f)


def _k0_body(dp_ref, x_ref, w_ref, g_ref, dis_ref):
    deg = (dp_ref[0] + dp_ref[1])[:, 0:1]
    row = lax.broadcasted_iota(jnp.int32, (NPAD, 1), 0)
    dis = jnp.where((deg > 0) & (row < N), lax.rsqrt(deg), 0.0)
    g_ref[...] = jnp.dot(x_ref[...], w_ref[...],
                         preferred_element_type=jnp.float32) * dis
    dis_ref[...] = dis


def _kmid_body(p_ref, dis_ref, b_ref, w_ref, g_ref):
    dis = dis_ref[...]
    z = jnp.maximum((p_ref[0] + p_ref[1]) * dis + b_ref[...], 0.0)
    g_ref[...] = jnp.dot(z, w_ref[...], preferred_element_type=jnp.float32) * dis


def _k4_body(p_ref, dis_ref, b_ref, g_ref):
    dis = dis_ref[...]
    g_ref[...] = jnp.maximum((p_ref[0] + p_ref[1]) * dis + b_ref[...], 0.0) * dis


def _k5_body(p_ref, dis_ref, b_ref, w_ref, o_ref):
    o_ref[...] = jnp.dot((p_ref[0] + p_ref[1]) * dis_ref[...], w_ref[...],
                         preferred_element_type=jnp.float32) + b_ref[...]


def _views(flat1d, chunk, nmax):
    rows = EPAD // chunk + nmax
    return lax.slice(flat1d, (0,), (rows * chunk,)).reshape(rows, chunk)


def kernel(x, edge_index, W1, b1, W2, b2, W3, b3, W4, b4, W5, b5):
    f32 = jnp.float32
    xp = jnp.concatenate([x, jnp.zeros((NPAD - N, x.shape[1]), f32)], axis=0)
    loop = jnp.arange(N, dtype=jnp.int32)
    padv = jnp.full((PAD1D - E_TOT,), N, dtype=jnp.int32)
    src1d = jnp.concatenate([edge_index[0].astype(jnp.int32), loop, padv])
    dst1d = jnp.concatenate([edge_index[1].astype(jnp.int32), loop, padv])
    src128, dst128 = _views(src1d, 128, 100), _views(dst1d, 128, 100)
    src_w = {128: _views(src1d, 64, 205), 64: src128, 32: src128, 16: src128}
    dst_w = {128: _views(dst1d, 64, 205), 64: dst128, 32: dst128, 16: dst128}

    deg_parts = _sc_degree(dst128)
    g, dis = pl.pallas_call(_k0_body, out_shape=(
        jax.ShapeDtypeStruct((NPAD, 128), f32),
        jax.ShapeDtypeStruct((NPAD, 1), f32)))(deg_parts, xp, W1)

    p = _sc_propagate(g, src_w[128], dst_w[128], 128)
    g = pl.pallas_call(_kmid_body, out_shape=jax.ShapeDtypeStruct(
        (NPAD, 64), f32))(p, dis, b1.reshape(1, -1), W2)
    p = _sc_propagate(g, src_w[64], dst_w[64], 64)
    g = pl.pallas_call(_kmid_body, out_shape=jax.ShapeDtypeStruct(
        (NPAD, 32), f32))(p, dis, b2.reshape(1, -1), W3)
    p = _sc_propagate(g, src_w[32], dst_w[32], 32)
    g = pl.pallas_call(_kmid_body, out_shape=jax.ShapeDtypeStruct(
        (NPAD, 16), f32))(p, dis, b3.reshape(1, -1), W4)
    p = _sc_propagate(g, src_w[16], dst_w[16], 16)
    g = pl.pallas_call(_k4_body, out_shape=jax.ShapeDtypeStruct(
        (NPAD, 16), f32))(p, dis, b4.reshape(1, -1))
    p = _sc_propagate(g, src_w[16], dst_w[16], 16)
    out = pl.pallas_call(_k5_body, out_shape=jax.ShapeDtypeStruct(
        (NPAD, 40), f32))(p, dis, b5.reshape(1, -1), W5)
    return out[:N]


# speedup vs baseline: 1.1817x; 1.0445x over previous
import functools

import jax
import jax.numpy as jnp
from jax import lax
from jax.experimental import pallas as pl
from jax.experimental.pallas import tpu as pltpu
from jax.experimental.pallas import tpu_sc as plsc

N = 10000
NPAD = 10240
E_RAW = 320000
E_TOT = E_RAW + N
NTILES = 32
EPAD = 331776
PAD1D = EPAD + 13504
RPT = NPAD // 16

_MESH = plsc.VectorSubcoreMesh(core_axis_name="c", subcore_axis_name="s")

_SPLIT = {128: (64, 211), 64: (128, 81), 32: (128, 82), 16: (128, 81)}


def _sc_degree(dst_r):
    nmax = 100

    @functools.partial(
        pl.kernel,
        out_type=jax.ShapeDtypeStruct((2, NPAD, 16), jnp.float32),
        mesh=_MESH,
        compiler_params=pltpu.CompilerParams(use_tc_tiling_on_sc=False),
        scratch_types=[
            pltpu.VMEM((nmax, 128), jnp.int32),
            pltpu.VMEM((128, 16), jnp.float32),
            pltpu.VMEM_SHARED((NPAD, 16), jnp.float32),
            pltpu.SemaphoreType.DMA,
        ],
    )
    def k(dst_hbm, out_hbm, idx_v, ones_v, acc, isem):
        c = lax.axis_index("c")
        s = lax.axis_index("s")
        start = s * 162 + c * 81
        pltpu.async_copy(dst_hbm.at[pl.ds(start, nmax)], idx_v, isem)

        def fill(val):
            def body(j, _):
                ones_v[j, :] = jnp.full((16,), val, jnp.float32)
                return 0
            return lax.fori_loop(0, 128, body, 0)

        fill(0.0)
        for t in range(RPT // 128):
            pltpu.sync_copy(ones_v, acc.at[pl.ds(s * RPT + t * 128, 128)])
        fill(1.0)
        plsc.subcore_barrier()
        pltpu.make_async_copy(dst_hbm.at[pl.ds(start, nmax)], idx_v, isem).wait()

        def body(j, _):
            pltpu.async_copy(ones_v, acc.at[idx_v.at[j]], isem, add=True)
            return 0

        lax.fori_loop(0, 81, body, 0)

        def drain(j, _):
            pltpu.make_async_copy(ones_v, acc.at[idx_v.at[0]], isem).wait()
            return 0

        lax.fori_loop(0, 81, drain, 0)
        plsc.subcore_barrier()
        pltpu.sync_copy(acc.at[pl.ds(s * RPT, RPT)],
                        out_hbm.at[c, pl.ds(s * RPT, RPT)])

    return k(dst_r)


def _sc_propagate(g, src_f, dst_f, width):
    chunk, n0 = _SPLIT[width]
    t16 = EPAD // chunk // 16
    n1 = t16 - n0
    nmax = 211 if chunk == 64 else 100
    from_spmem = width <= 64

    @functools.partial(
        pl.kernel,
        out_type=jax.ShapeDtypeStruct((2, NPAD, width), jnp.float32),
        mesh=_MESH,
        compiler_params=pltpu.CompilerParams(use_tc_tiling_on_sc=False),
        scratch_types=[
            pltpu.VMEM((nmax, chunk), jnp.int32),
            pltpu.VMEM((nmax, chunk), jnp.int32),
            pltpu.VMEM((2, chunk, width), jnp.float32),
            pltpu.VMEM_SHARED((NPAD, width), jnp.float32),
            pltpu.VMEM_SHARED((NPAD if from_spmem else 16, width), jnp.float32),
            pltpu.SemaphoreType.DMA((2,)),
            pltpu.SemaphoreType.DMA((2,)),
            pltpu.SemaphoreType.DMA,
        ],
    )
    def k(g_hbm, src_hbm, dst_hbm, out_hbm, src_v, dst_v, rows_v, acc, table,
          gsem, ssem, isem):
        c = lax.axis_index("c")
        s = lax.axis_index("s")
        start = s * t16 + c * n0
        cnt = jnp.where(c == 0, n0, n1)
        pltpu.async_copy(src_hbm.at[pl.ds(start, nmax)], src_v, isem)
        pltpu.async_copy(dst_hbm.at[pl.ds(start, nmax)], dst_v, isem)
        if from_spmem:
            pltpu.async_copy(g_hbm.at[pl.ds(s * RPT, RPT)],
                             table.at[pl.ds(s * RPT, RPT)], isem)
        g_src = table if from_spmem else g_hbm

        zeros16 = jnp.zeros((16,), jnp.float32)

        def zbody(i, _):
            for kk in range(width // 16):
                rows_v[0, i, pl.ds(kk * 16, 16)] = zeros16
            return 0

        lax.fori_loop(0, chunk, zbody, 0)
        for t in range(RPT // chunk):
            pltpu.sync_copy(rows_v.at[0], acc.at[pl.ds(s * RPT + t * chunk, chunk)])
        pltpu.make_async_copy(src_hbm.at[pl.ds(start, nmax)], src_v, isem).wait()
        pltpu.make_async_copy(dst_hbm.at[pl.ds(start, nmax)], dst_v, isem).wait()
        if from_spmem:
            pltpu.make_async_copy(g_hbm.at[pl.ds(s * RPT, RPT)],
                                  table.at[pl.ds(s * RPT, RPT)], isem).wait()
        plsc.subcore_barrier()

        pltpu.async_copy(g_src.at[src_v.at[0]], rows_v.at[0], gsem.at[0])

        def body(j, _):
            par = lax.rem(j, 2)
            npar = lax.rem(j + 1, 2)

            if from_spmem:
                @pl.when(j >= 1)
                def _():
                    pltpu.make_async_copy(rows_v.at[npar], acc.at[dst_v.at[0]],
                                          ssem.at[npar]).wait()

            @pl.when(j < cnt - 1)
            def _():
                pltpu.async_copy(g_src.at[src_v.at[j + 1]], rows_v.at[npar],
                                 gsem.at[npar])

            pltpu.make_async_copy(g_src.at[src_v.at[j]], rows_v.at[par],
                                  gsem.at[par]).wait()
            if from_spmem:
                pltpu.async_copy(rows_v.at[par], acc.at[dst_v.at[j]],
                                 ssem.at[par], add=True)
            else:
                pltpu.sync_copy(rows_v.at[par], acc.at[dst_v.at[j]], add=True)
            return 0

        lax.fori_loop(0, cnt, body, 0)
        if from_spmem:
            last = lax.rem(cnt - 1, 2)
            pltpu.make_async_copy(rows_v.at[last], acc.at[dst_v.at[0]],
                                  ssem.at[last]).wait()
        plsc.subcore_barrier()
        pltpu.sync_copy(acc.at[pl.ds(s * RPT, RPT)],
                        out_hbm.at[c, pl.ds(s * RPT, RPT)])

    return k(g, src_f, dst_f)


def _k0_body(dp_ref, x_ref, w_ref, g_ref, dis_ref):
    deg = (dp_ref[0] + dp_ref[1])[:, 0:1]
    row = lax.broadcasted_iota(jnp.int32, (NPAD, 1), 0)
    dis = jnp.where((deg > 0) & (row < N), lax.rsqrt(deg), 0.0)
    g_ref[...] = jnp.dot(x_ref[...], w_ref[...],
                         preferred_element_type=jnp.float32) * dis
    dis_ref[...] = dis


def _kmid_body(p_ref, dis_ref, b_ref, w_ref, g_ref):
    dis = dis_ref[...]
    z = jnp.maximum((p_ref[0] + p_ref[1]) * dis + b_ref[...], 0.0)
    g_ref[...] = jnp.dot(z, w_ref[...], preferred_element_type=jnp.float32) * dis


def _k4_body(p_ref, dis_ref, b_ref, g_ref):
    dis = dis_ref[...]
    g_ref[...] = jnp.maximum((p_ref[0] + p_ref[1]) * dis + b_ref[...], 0.0) * dis


def _k5_body(p_ref, dis_ref, b_ref, w_ref, o_ref):
    o_ref[...] = jnp.dot((p_ref[0] + p_ref[1]) * dis_ref[...], w_ref[...],
                         preferred_element_type=jnp.float32) + b_ref[...]


def _views(flat1d, chunk, nmax):
    rows = EPAD // chunk + nmax
    return lax.slice(flat1d, (0,), (rows * chunk,)).reshape(rows, chunk)


def kernel(x, edge_index, W1, b1, W2, b2, W3, b3, W4, b4, W5, b5):
    f32 = jnp.float32
    xp = jnp.concatenate([x, jnp.zeros((NPAD - N, x.shape[1]), f32)], axis=0)
    loop = jnp.arange(N, dtype=jnp.int32)
    padv = jnp.full((PAD1D - E_TOT,), N, dtype=jnp.int32)
    src1d = jnp.concatenate([edge_index[0].astype(jnp.int32), loop, padv])
    dst1d = jnp.concatenate([edge_index[1].astype(jnp.int32), loop, padv])
    src128, dst128 = _views(src1d, 128, 100), _views(dst1d, 128, 100)
    src_w = {128: _views(src1d, 64, 211), 64: src128, 32: src128, 16: src128}
    dst_w = {128: _views(dst1d, 64, 211), 64: dst128, 32: dst128, 16: dst128}

    deg_parts = _sc_degree(dst128)
    g, dis = pl.pallas_call(_k0_body, out_shape=(
        jax.ShapeDtypeStruct((NPAD, 128), f32),
        jax.ShapeDtypeStruct((NPAD, 1), f32)))(deg_parts, xp, W1)

    p = _sc_propagate(g, src_w[128], dst_w[128], 128)
    g = pl.pallas_call(_kmid_body, out_shape=jax.ShapeDtypeStruct(
        (NPAD, 64), f32))(p, dis, b1.reshape(1, -1), W2)
    p = _sc_propagate(g, src_w[64], dst_w[64], 64)
    g = pl.pallas_call(_kmid_body, out_shape=jax.ShapeDtypeStruct(
        (NPAD, 32), f32))(p, dis, b2.reshape(1, -1), W3)
    p = _sc_propagate(g, src_w[32], dst_w[32], 32)
    g = pl.pallas_call(_kmid_body, out_shape=jax.ShapeDtypeStruct(
        (NPAD, 16), f32))(p, dis, b3.reshape(1, -1), W4)
    p = _sc_propagate(g, src_w[16], dst_w[16], 16)
    g = pl.pallas_call(_k4_body, out_shape=jax.ShapeDtypeStruct(
        (NPAD, 16), f32))(p, dis, b4.reshape(1, -1))
    p = _sc_propagate(g, src_w[16], dst_w[16], 16)
    out = pl.pallas_call(_k5_body, out_shape=jax.ShapeDtypeStruct(
        (NPAD, 40), f32))(p, dis, b5.reshape(1, -1), W5)
    return out[:N]


# speedup vs baseline: 1.1840x; 1.0019x over previous
import functools

import jax
import jax.numpy as jnp
from jax import lax
from jax.experimental import pallas as pl
from jax.experimental.pallas import tpu as pltpu
from jax.experimental.pallas import tpu_sc as plsc

N = 10000
NPAD = 10240
E_RAW = 320000
E_TOT = E_RAW + N
NTILES = 32
EPAD = 331776
PAD1D = EPAD + 12800
RPT = NPAD // 16

_MESH = plsc.VectorSubcoreMesh(core_axis_name="c", subcore_axis_name="s")

_SPLIT = {128: (128, 81), 64: (128, 81), 32: (128, 82), 16: (128, 81)}


def _sc_degree(dst_r):
    nmax = 100

    @functools.partial(
        pl.kernel,
        out_type=jax.ShapeDtypeStruct((2, NPAD, 16), jnp.float32),
        mesh=_MESH,
        compiler_params=pltpu.CompilerParams(use_tc_tiling_on_sc=False),
        scratch_types=[
            pltpu.VMEM((nmax, 128), jnp.int32),
            pltpu.VMEM((128, 16), jnp.float32),
            pltpu.VMEM_SHARED((NPAD, 16), jnp.float32),
            pltpu.SemaphoreType.DMA,
        ],
    )
    def k(dst_hbm, out_hbm, idx_v, ones_v, acc, isem):
        c = lax.axis_index("c")
        s = lax.axis_index("s")
        start = s * 162 + c * 81
        pltpu.async_copy(dst_hbm.at[pl.ds(start, nmax)], idx_v, isem)

        def fill(val):
            def body(j, _):
                ones_v[j, :] = jnp.full((16,), val, jnp.float32)
                return 0
            return lax.fori_loop(0, 128, body, 0)

        fill(0.0)
        for t in range(RPT // 128):
            pltpu.sync_copy(ones_v, acc.at[pl.ds(s * RPT + t * 128, 128)])
        fill(1.0)
        plsc.subcore_barrier()
        pltpu.make_async_copy(dst_hbm.at[pl.ds(start, nmax)], idx_v, isem).wait()

        def body(j, _):
            pltpu.async_copy(ones_v, acc.at[idx_v.at[j]], isem, add=True)
            return 0

        lax.fori_loop(0, 81, body, 0)

        def drain(j, _):
            pltpu.make_async_copy(ones_v, acc.at[idx_v.at[0]], isem).wait()
            return 0

        lax.fori_loop(0, 81, drain, 0)
        plsc.subcore_barrier()
        pltpu.sync_copy(acc.at[pl.ds(s * RPT, RPT)],
                        out_hbm.at[c, pl.ds(s * RPT, RPT)])

    return k(dst_r)


def _sc_propagate(g, src_f, dst_f, width):
    chunk, n0 = _SPLIT[width]
    t16 = EPAD // chunk // 16
    n1 = t16 - n0
    nmax = 100
    halves = 2 if width == 128 else 1
    hw = width // halves
    out_t = ((2, 2, NPAD, hw) if halves == 2 else (2, NPAD, width))

    @functools.partial(
        pl.kernel,
        out_type=jax.ShapeDtypeStruct(out_t, jnp.float32),
        mesh=_MESH,
        compiler_params=pltpu.CompilerParams(use_tc_tiling_on_sc=False),
        scratch_types=[
            pltpu.VMEM((nmax, chunk), jnp.int32),
            pltpu.VMEM((nmax, chunk), jnp.int32),
            pltpu.VMEM((2, chunk, hw), jnp.float32),
            pltpu.VMEM_SHARED((NPAD, hw), jnp.float32),
            pltpu.VMEM_SHARED((NPAD, hw), jnp.float32),
            pltpu.SemaphoreType.DMA((2,)),
            pltpu.SemaphoreType.DMA((2,)),
            pltpu.SemaphoreType.DMA,
        ],
    )
    def k(g_hbm, src_hbm, dst_hbm, out_hbm, src_v, dst_v, rows_v, acc, table,
          gsem, ssem, isem):
        c = lax.axis_index("c")
        s = lax.axis_index("s")
        start = s * t16 + c * n0
        cnt = jnp.where(c == 0, n0, n1)
        strip = pl.ds(s * RPT, RPT)

        def gslab(h):
            return g_hbm.at[h, strip] if halves == 2 else g_hbm.at[strip]

        pltpu.async_copy(src_hbm.at[pl.ds(start, nmax)], src_v, isem)
        pltpu.async_copy(dst_hbm.at[pl.ds(start, nmax)], dst_v, isem)
        pltpu.async_copy(gslab(0), table.at[strip], isem)

        zeros16 = jnp.zeros((16,), jnp.float32)

        def zero_acc():
            def zbody(i, _):
                for kk in range(hw // 16):
                    rows_v[0, i, pl.ds(kk * 16, 16)] = zeros16
                return 0

            lax.fori_loop(0, chunk, zbody, 0)
            for t in range(RPT // chunk):
                pltpu.sync_copy(rows_v.at[0],
                                acc.at[pl.ds(s * RPT + t * chunk, chunk)])

        def run_half(h):
            pltpu.async_copy(table.at[src_v.at[0]], rows_v.at[0], gsem.at[0])

            def body(j, _):
                par = lax.rem(j, 2)
                npar = lax.rem(j + 1, 2)

                @pl.when(j >= 1)
                def _():
                    pltpu.make_async_copy(rows_v.at[npar], acc.at[dst_v.at[0]],
                                          ssem.at[npar]).wait()

                @pl.when(j < cnt - 1)
                def _():
                    pltpu.async_copy(table.at[src_v.at[j + 1]], rows_v.at[npar],
                                     gsem.at[npar])

                pltpu.make_async_copy(table.at[src_v.at[j]], rows_v.at[par],
                                      gsem.at[par]).wait()
                pltpu.async_copy(rows_v.at[par], acc.at[dst_v.at[j]],
                                 ssem.at[par], add=True)
                return 0

            lax.fori_loop(0, cnt, body, 0)
            last = lax.rem(cnt - 1, 2)
            pltpu.make_async_copy(rows_v.at[last], acc.at[dst_v.at[0]],
                                  ssem.at[last]).wait()
            plsc.subcore_barrier()
            if halves == 2:
                pltpu.sync_copy(acc.at[strip], out_hbm.at[h, c, strip])
            else:
                pltpu.sync_copy(acc.at[strip], out_hbm.at[c, strip])

        zero_acc()
        pltpu.make_async_copy(src_hbm.at[pl.ds(start, nmax)], src_v, isem).wait()
        pltpu.make_async_copy(dst_hbm.at[pl.ds(start, nmax)], dst_v, isem).wait()
        pltpu.make_async_copy(gslab(0), table.at[strip], isem).wait()
        plsc.subcore_barrier()
        run_half(0)
        if halves == 2:
            pltpu.async_copy(gslab(1), table.at[strip], isem)
            zero_acc()
            pltpu.make_async_copy(gslab(1), table.at[strip], isem).wait()
            plsc.subcore_barrier()
            run_half(1)

    return k(g, src_f, dst_f)


def _k0_body(dp_ref, x_ref, w_ref, g_ref, dis_ref):
    deg = (dp_ref[0] + dp_ref[1])[:, 0:1]
    row = lax.broadcasted_iota(jnp.int32, (NPAD, 1), 0)
    dis = jnp.where((deg > 0) & (row < N), lax.rsqrt(deg), 0.0)
    h1 = jnp.dot(x_ref[...], w_ref[...],
                 preferred_element_type=jnp.float32) * dis
    g_ref[0] = h1[:, 0:64]
    g_ref[1] = h1[:, 64:128]
    dis_ref[...] = dis


def _k1_body(p_ref, dis_ref, b_ref, w_ref, g_ref):
    dis = dis_ref[...]
    za = p_ref[0, 0] + p_ref[0, 1]
    zb = p_ref[1, 0] + p_ref[1, 1]
    z = jnp.maximum(jnp.concatenate([za, zb], axis=1) * dis + b_ref[...], 0.0)
    g_ref[...] = jnp.dot(z, w_ref[...], preferred_element_type=jnp.float32) * dis


def _kmid_body(p_ref, dis_ref, b_ref, w_ref, g_ref):
    dis = dis_ref[...]
    z = jnp.maximum((p_ref[0] + p_ref[1]) * dis + b_ref[...], 0.0)
    g_ref[...] = jnp.dot(z, w_ref[...], preferred_element_type=jnp.float32) * dis


def _k4_body(p_ref, dis_ref, b_ref, g_ref):
    dis = dis_ref[...]
    g_ref[...] = jnp.maximum((p_ref[0] + p_ref[1]) * dis + b_ref[...], 0.0) * dis


def _k5_body(p_ref, dis_ref, b_ref, w_ref, o_ref):
    o_ref[...] = jnp.dot((p_ref[0] + p_ref[1]) * dis_ref[...], w_ref[...],
                         preferred_element_type=jnp.float32) + b_ref[...]


def _views(flat1d, chunk, nmax):
    rows = EPAD // chunk + nmax
    return lax.slice(flat1d, (0,), (rows * chunk,)).reshape(rows, chunk)


def kernel(x, edge_index, W1, b1, W2, b2, W3, b3, W4, b4, W5, b5):
    f32 = jnp.float32
    xp = jnp.concatenate([x, jnp.zeros((NPAD - N, x.shape[1]), f32)], axis=0)
    loop = jnp.arange(N, dtype=jnp.int32)
    padv = jnp.full((PAD1D - E_TOT,), N, dtype=jnp.int32)
    src1d = jnp.concatenate([edge_index[0].astype(jnp.int32), loop, padv])
    dst1d = jnp.concatenate([edge_index[1].astype(jnp.int32), loop, padv])
    src128, dst128 = _views(src1d, 128, 100), _views(dst1d, 128, 100)

    deg_parts = _sc_degree(dst128)
    g, dis = pl.pallas_call(_k0_body, out_shape=(
        jax.ShapeDtypeStruct((2, NPAD, 64), f32),
        jax.ShapeDtypeStruct((NPAD, 1), f32)))(deg_parts, xp, W1)

    p = _sc_propagate(g, src128, dst128, 128)
    g = pl.pallas_call(_k1_body, out_shape=jax.ShapeDtypeStruct(
        (NPAD, 64), f32))(p, dis, b1.reshape(1, -1), W2)
    p = _sc_propagate(g, src128, dst128, 64)
    g = pl.pallas_call(_kmid_body, out_shape=jax.ShapeDtypeStruct(
        (NPAD, 32), f32))(p, dis, b2.reshape(1, -1), W3)
    p = _sc_propagate(g, src128, dst128, 32)
    g = pl.pallas_call(_kmid_body, out_shape=jax.ShapeDtypeStruct(
        (NPAD, 16), f32))(p, dis, b3.reshape(1, -1), W4)
    p = _sc_propagate(g, src128, dst128, 16)
    g = pl.pallas_call(_k4_body, out_shape=jax.ShapeDtypeStruct(
        (NPAD, 16), f32))(p, dis, b4.reshape(1, -1))
    p = _sc_propagate(g, src128, dst128, 16)
    out = pl.pallas_call(_k5_body, out_shape=jax.ShapeDtypeStruct(
        (NPAD, 40), f32))(p, dis, b5.reshape(1, -1), W5)
    return out[:N]


# speedup vs baseline: 1.2599x; 1.0641x over previous
import functools

import jax
import jax.numpy as jnp
from jax import lax
from jax.experimental import pallas as pl
from jax.experimental.pallas import tpu as pltpu
from jax.experimental.pallas import tpu_sc as plsc

N = 10000
NPAD = 10240
E_RAW = 320000
E_TOT = E_RAW + N
NTILES = 32
EPAD = 331776
PAD1D = EPAD + 12800
RPT = NPAD // 16

_MESH = plsc.VectorSubcoreMesh(core_axis_name="c", subcore_axis_name="s")

_SPLIT = {128: (128, 81), 64: (128, 81), 32: (128, 82), 16: (128, 81)}


def _sc_degree(dst_r):
    nmax = 100

    @functools.partial(
        pl.kernel,
        out_type=jax.ShapeDtypeStruct((2, NPAD, 16), jnp.float32),
        mesh=_MESH,
        compiler_params=pltpu.CompilerParams(use_tc_tiling_on_sc=False),
        scratch_types=[
            pltpu.VMEM((nmax, 128), jnp.int32),
            pltpu.VMEM((128, 16), jnp.float32),
            pltpu.VMEM_SHARED((NPAD, 16), jnp.float32),
            pltpu.SemaphoreType.DMA,
        ],
    )
    def k(dst_hbm, out_hbm, idx_v, ones_v, acc, isem):
        c = lax.axis_index("c")
        s = lax.axis_index("s")
        start = s * 162 + c * 81
        pltpu.async_copy(dst_hbm.at[pl.ds(start, nmax)], idx_v, isem)

        def fill(val):
            def body(j, _):
                ones_v[j, :] = jnp.full((16,), val, jnp.float32)
                return 0
            return lax.fori_loop(0, 128, body, 0)

        fill(0.0)
        for t in range(RPT // 128):
            pltpu.sync_copy(ones_v, acc.at[pl.ds(s * RPT + t * 128, 128)])
        fill(1.0)
        plsc.subcore_barrier()
        pltpu.make_async_copy(dst_hbm.at[pl.ds(start, nmax)], idx_v, isem).wait()

        def body(j, _):
            pltpu.async_copy(ones_v, acc.at[idx_v.at[j]], isem, add=True)
            return 0

        lax.fori_loop(0, 81, body, 0)

        def drain(j, _):
            pltpu.make_async_copy(ones_v, acc.at[idx_v.at[0]], isem).wait()
            return 0

        lax.fori_loop(0, 81, drain, 0)
        plsc.subcore_barrier()
        pltpu.sync_copy(acc.at[pl.ds(s * RPT, RPT)],
                        out_hbm.at[c, pl.ds(s * RPT, RPT)])

    return k(dst_r)


def _sc_propagate(g, src_f, dst_f, width):
    chunk, n0 = _SPLIT[width]
    t16 = EPAD // chunk // 16
    n1 = t16 - n0
    nmax = 100
    halves = 2 if width == 128 else 1
    hw = width // halves
    out_t = (2, NPAD, width)

    @functools.partial(
        pl.kernel,
        out_type=jax.ShapeDtypeStruct(out_t, jnp.float32),
        mesh=_MESH,
        compiler_params=pltpu.CompilerParams(use_tc_tiling_on_sc=False),
        scratch_types=[
            pltpu.VMEM((nmax, chunk), jnp.int32),
            pltpu.VMEM((nmax, chunk), jnp.int32),
            pltpu.VMEM((2, chunk, hw), jnp.float32),
            pltpu.VMEM_SHARED((NPAD, hw), jnp.float32),
            pltpu.VMEM_SHARED((NPAD, hw), jnp.float32),
            pltpu.SemaphoreType.DMA((2,)),
            pltpu.SemaphoreType.DMA((2,)),
            pltpu.SemaphoreType.DMA,
        ],
    )
    def k(g_hbm, src_hbm, dst_hbm, out_hbm, src_v, dst_v, rows_v, acc, table,
          gsem, ssem, isem):
        c = lax.axis_index("c")
        s = lax.axis_index("s")
        start = s * t16 + c * n0
        cnt = jnp.where(c == 0, n0, n1)
        strip = pl.ds(s * RPT, RPT)

        def gslab(h):
            return (g_hbm.at[strip, pl.ds(h * hw, hw)] if halves == 2
                    else g_hbm.at[strip])

        pltpu.async_copy(src_hbm.at[pl.ds(start, nmax)], src_v, isem)
        pltpu.async_copy(dst_hbm.at[pl.ds(start, nmax)], dst_v, isem)
        pltpu.async_copy(gslab(0), table.at[strip], isem)

        zeros16 = jnp.zeros((16,), jnp.float32)

        def zero_acc():
            def zbody(i, _):
                for kk in range(hw // 16):
                    rows_v[0, i, pl.ds(kk * 16, 16)] = zeros16
                return 0

            lax.fori_loop(0, chunk, zbody, 0)
            for t in range(RPT // chunk):
                pltpu.sync_copy(rows_v.at[0],
                                acc.at[pl.ds(s * RPT + t * chunk, chunk)])

        def run_half(h):
            pltpu.async_copy(table.at[src_v.at[0]], rows_v.at[0], gsem.at[0])

            def body(j, _):
                par = lax.rem(j, 2)
                npar = lax.rem(j + 1, 2)

                @pl.when(j >= 1)
                def _():
                    pltpu.make_async_copy(rows_v.at[npar], acc.at[dst_v.at[0]],
                                          ssem.at[npar]).wait()

                @pl.when(j < cnt - 1)
                def _():
                    pltpu.async_copy(table.at[src_v.at[j + 1]], rows_v.at[npar],
                                     gsem.at[npar])

                pltpu.make_async_copy(table.at[src_v.at[j]], rows_v.at[par],
                                      gsem.at[par]).wait()
                pltpu.async_copy(rows_v.at[par], acc.at[dst_v.at[j]],
                                 ssem.at[par], add=True)
                return 0

            lax.fori_loop(0, cnt, body, 0)
            last = lax.rem(cnt - 1, 2)
            pltpu.make_async_copy(rows_v.at[last], acc.at[dst_v.at[0]],
                                  ssem.at[last]).wait()
            plsc.subcore_barrier()
            if halves == 2:
                pltpu.sync_copy(acc.at[strip],
                                out_hbm.at[c, strip, pl.ds(h * hw, hw)])
            else:
                pltpu.sync_copy(acc.at[strip], out_hbm.at[c, strip])

        zero_acc()
        pltpu.make_async_copy(src_hbm.at[pl.ds(start, nmax)], src_v, isem).wait()
        pltpu.make_async_copy(dst_hbm.at[pl.ds(start, nmax)], dst_v, isem).wait()
        pltpu.make_async_copy(gslab(0), table.at[strip], isem).wait()
        plsc.subcore_barrier()
        run_half(0)
        if halves == 2:
            pltpu.async_copy(gslab(1), table.at[strip], isem)
            zero_acc()
            pltpu.make_async_copy(gslab(1), table.at[strip], isem).wait()
            plsc.subcore_barrier()
            run_half(1)

    return k(g, src_f, dst_f)


def _k0_body(dp_ref, x_ref, w_ref, g_ref, dis_ref):
    deg = (dp_ref[0] + dp_ref[1])[:, 0:1]
    row = lax.broadcasted_iota(jnp.int32, (NPAD, 1), 0)
    dis = jnp.where((deg > 0) & (row < N), lax.rsqrt(deg), 0.0)
    g_ref[...] = jnp.dot(x_ref[...], w_ref[...],
                         preferred_element_type=jnp.float32) * dis
    dis_ref[...] = dis


def _kmid_body(p_ref, dis_ref, b_ref, w_ref, g_ref):
    dis = dis_ref[...]
    z = jnp.maximum((p_ref[0] + p_ref[1]) * dis + b_ref[...], 0.0)
    g_ref[...] = jnp.dot(z, w_ref[...], preferred_element_type=jnp.float32) * dis


def _k4_body(p_ref, dis_ref, b_ref, g_ref):
    dis = dis_ref[...]
    g_ref[...] = jnp.maximum((p_ref[0] + p_ref[1]) * dis + b_ref[...], 0.0) * dis


def _k5_body(p_ref, dis_ref, b_ref, w_ref, o_ref):
    o_ref[...] = jnp.dot((p_ref[0] + p_ref[1]) * dis_ref[...], w_ref[...],
                         preferred_element_type=jnp.float32) + b_ref[...]


def _views(flat1d, chunk, nmax):
    rows = EPAD // chunk + nmax
    return lax.slice(flat1d, (0,), (rows * chunk,)).reshape(rows, chunk)


def kernel(x, edge_index, W1, b1, W2, b2, W3, b3, W4, b4, W5, b5):
    f32 = jnp.float32
    xp = jnp.concatenate([x, jnp.zeros((NPAD - N, x.shape[1]), f32)], axis=0)
    loop = jnp.arange(N, dtype=jnp.int32)
    padv = jnp.full((PAD1D - E_TOT,), N, dtype=jnp.int32)
    src1d = jnp.concatenate([edge_index[0].astype(jnp.int32), loop, padv])
    dst1d = jnp.concatenate([edge_index[1].astype(jnp.int32), loop, padv])
    src128, dst128 = _views(src1d, 128, 100), _views(dst1d, 128, 100)

    deg_parts = _sc_degree(dst128)
    g, dis = pl.pallas_call(_k0_body, out_shape=(
        jax.ShapeDtypeStruct((NPAD, 128), f32),
        jax.ShapeDtypeStruct((NPAD, 1), f32)))(deg_parts, xp, W1)

    p = _sc_propagate(g, src128, dst128, 128)
    g = pl.pallas_call(_kmid_body, out_shape=jax.ShapeDtypeStruct(
        (NPAD, 64), f32))(p, dis, b1.reshape(1, -1), W2)
    p = _sc_propagate(g, src128, dst128, 64)
    g = pl.pallas_call(_kmid_body, out_shape=jax.ShapeDtypeStruct(
        (NPAD, 32), f32))(p, dis, b2.reshape(1, -1), W3)
    p = _sc_propagate(g, src128, dst128, 32)
    g = pl.pallas_call(_kmid_body, out_shape=jax.ShapeDtypeStruct(
        (NPAD, 16), f32))(p, dis, b3.reshape(1, -1), W4)
    p = _sc_propagate(g, src128, dst128, 16)
    g = pl.pallas_call(_k4_body, out_shape=jax.ShapeDtypeStruct(
        (NPAD, 16), f32))(p, dis, b4.reshape(1, -1))
    p = _sc_propagate(g, src128, dst128, 16)
    out = pl.pallas_call(_k5_body, out_shape=jax.ShapeDtypeStruct(
        (NPAD, 40), f32))(p, dis, b5.reshape(1, -1), W5)
    return out[:N]
